# baseline (speedup 1.0000x reference)
import jax
import jax.numpy as jnp
from jax.experimental import pallas as pl
from jax.experimental.pallas import tpu as pltpu

_BT = 2048
_KSPLIT = 2
_E = 16
_NEG = -3.0e38


def _top2(logits):
    eidx = jax.lax.broadcasted_iota(jnp.int32, logits.shape, 1)
    m1 = jnp.max(logits, axis=1, keepdims=True)
    i1 = jnp.min(jnp.where(logits == m1, eidx, _E), axis=1, keepdims=True)
    masked = jnp.where(eidx == i1, _NEG, logits)
    m2 = jnp.max(masked, axis=1, keepdims=True)
    i2 = jnp.min(jnp.where(masked == m2, eidx, _E), axis=1, keepdims=True)
    z = jnp.exp(m2 - m1)
    w1 = 1.0 / (1.0 + z)
    return (jnp.concatenate([w1, z * w1], axis=1),
            jnp.concatenate([i1, i2], axis=1))


def _router_body(x_ref, w_ref, wout_ref, iout_ref, acc_ref):
    j = pl.program_id(1)
    partial = jax.lax.dot_general(
        x_ref[...], w_ref[...],
        dimension_numbers=(((1,), (1,)), ((), ())),
        preferred_element_type=jnp.float32,
    )

    @pl.when(j == 0)
    def _init():
        acc_ref[...] = partial

    @pl.when(j == _KSPLIT - 1)
    def _fin():
        logits = acc_ref[...] + partial
        w, idx = _top2(logits)
        wout_ref[...] = w
        iout_ref[...] = idx

    @pl.when(jnp.logical_and(j > 0, j < _KSPLIT - 1))
    def _acc():
        acc_ref[...] = acc_ref[...] + partial


@jax.jit
def _route(x2d, W):
    nt, d = x2d.shape
    kb = d // _KSPLIT
    return pl.pallas_call(
        _router_body,
        grid=(nt // _BT, _KSPLIT),
        in_specs=[
            pl.BlockSpec((_BT, kb), lambda i, j: (i, j)),
            pl.BlockSpec((_E, kb), lambda i, j: (0, j)),
        ],
        out_specs=[
            pl.BlockSpec((_BT, 2), lambda i, j: (i, 0)),
            pl.BlockSpec((_BT, 2), lambda i, j: (i, 0)),
        ],
        out_shape=[
            jax.ShapeDtypeStruct((nt, 2), jnp.float32),
            jax.ShapeDtypeStruct((nt, 2), jnp.int32),
        ],
        scratch_shapes=[
            pltpu.VMEM((_BT, _E), jnp.float32),
        ],
        compiler_params=pltpu.CompilerParams(
            dimension_semantics=("arbitrary", "arbitrary"),
        ),
    )(x2d, W)


def kernel(x, W):
    B, T, D = x.shape
    w, i = _route(x.reshape(B * T, D), W)
    return w.reshape(B, T, 2), i.reshape(B, T, 2)
